# Initial kernel scaffold; baseline (speedup 1.0000x reference)
import jax
import jax.numpy as jnp
from jax.experimental import pallas as pl


def kernel(x_ids, table):
    raise NotImplementedError("write your pallas kernel here")


# speedup vs baseline: 1.4590x; 1.4590x over previous
import jax
import jax.numpy as jnp
from jax import lax
from jax.experimental import pallas as pl
from jax.experimental.pallas import tpu as pltpu
from jax.experimental.pallas import tpu_sc as plsc

_EMBED = 32
_NW = 32
_CHUNK = 1024


def _emb_body(idx_hbm, table_hbm, out_hbm, idx_v, rows_v, sem):
    n = idx_hbm.shape[0]
    per_w = n // _NW
    wid = lax.axis_index("s") * 2 + lax.axis_index("c")
    base = wid * per_w

    def step(i, carry):
        off = base + i * _CHUNK
        pltpu.sync_copy(idx_hbm.at[pl.ds(off, _CHUNK)], idx_v)
        pltpu.async_copy(table_hbm.at[idx_v], rows_v, sem).wait()
        pltpu.sync_copy(rows_v, out_hbm.at[pl.ds(off, _CHUNK)])
        return carry

    lax.fori_loop(0, per_w // _CHUNK, step, 0)


def kernel(x_ids, table):
    b, h = x_ids.shape
    n = b * h
    idx = x_ids.reshape(n).astype(jnp.int32)
    run = pl.kernel(
        _emb_body,
        mesh=plsc.VectorSubcoreMesh(core_axis_name="c", subcore_axis_name="s"),
        out_type=jax.ShapeDtypeStruct((n, _EMBED), jnp.float32),
        scratch_types=[
            pltpu.VMEM((_CHUNK,), jnp.int32),
            pltpu.VMEM((_CHUNK, _EMBED), jnp.float32),
            pltpu.SemaphoreType.DMA,
        ],
        compiler_params=pltpu.CompilerParams(use_tc_tiling_on_sc=False),
    )
    out = run(idx, table)
    return out.reshape(b, h, _EMBED)


# speedup vs baseline: 1.4956x; 1.0251x over previous
import jax
import jax.numpy as jnp
from jax import lax
from jax.experimental import pallas as pl
from jax.experimental.pallas import tpu as pltpu
from jax.experimental.pallas import tpu_sc as plsc

_EMBED = 32
_NW = 32
_CHUNK = 1280
_NBUF = 2


def _emb_body(idx_hbm, table_hbm, out_hbm, idx_v, rows_v, sem_g, sem_o):
    n = idx_hbm.shape[0]
    per_w = n // _NW
    wid = lax.axis_index("s") * 2 + lax.axis_index("c")
    base = wid * per_w
    nchunks = per_w // _CHUNK

    pltpu.sync_copy(idx_hbm.at[pl.ds(base, per_w)], idx_v)

    def gather(c, b):
        return pltpu.make_async_copy(
            table_hbm.at[idx_v.at[pl.ds(c * _CHUNK, _CHUNK)]],
            rows_v.at[b], sem_g.at[b])

    def writeback(c, b):
        return pltpu.make_async_copy(
            rows_v.at[b], out_hbm.at[pl.ds(base + c * _CHUNK, _CHUNK)],
            sem_o.at[b])

    for b in range(_NBUF):
        gather(b, b).start()
    for b in range(_NBUF):
        gather(b, b).wait()
        writeback(b, b).start()

    def step(g, carry):
        for b in range(_NBUF):
            c = _NBUF * g + b
            writeback(c - _NBUF, b).wait()
            gather(c, b).start()
            gather(c, b).wait()
            writeback(c, b).start()
        return carry

    lax.fori_loop(1, nchunks // _NBUF, step, 0)

    for b in range(_NBUF):
        writeback(nchunks - _NBUF + b, b).wait()


def kernel(x_ids, table):
    b, h = x_ids.shape
    n = b * h
    idx = x_ids.reshape(n).astype(jnp.int32)
    run = pl.kernel(
        _emb_body,
        mesh=plsc.VectorSubcoreMesh(core_axis_name="c", subcore_axis_name="s"),
        out_type=jax.ShapeDtypeStruct((n, _EMBED), jnp.float32),
        scratch_types=[
            pltpu.VMEM((n // _NW,), jnp.int32),
            pltpu.VMEM((_NBUF, _CHUNK, _EMBED), jnp.float32),
            pltpu.SemaphoreType.DMA((_NBUF,)),
            pltpu.SemaphoreType.DMA((_NBUF,)),
        ],
        compiler_params=pltpu.CompilerParams(use_tc_tiling_on_sc=False),
    )
    out = run(idx, table)
    return out.reshape(b, h, _EMBED)


# speedup vs baseline: 1.4971x; 1.0010x over previous
import jax
import jax.numpy as jnp
from jax import lax
from jax.experimental import pallas as pl
from jax.experimental.pallas import tpu as pltpu
from jax.experimental.pallas import tpu_sc as plsc

_EMBED = 32
_NW = 32
_CHUNK = 1280
_NBUF = 2


def _emb_body(idx_hbm, table_hbm, out_hbm, idx_v, rows_v, sem_g, sem_o):
    n = idx_hbm.shape[0]
    per_w = n // _NW
    wid = lax.axis_index("s") * 2 + lax.axis_index("c")
    base = wid * per_w
    nchunks = per_w // _CHUNK

    pltpu.sync_copy(idx_hbm.at[pl.ds(base, per_w)], idx_v)

    def gather(c, b):
        return pltpu.make_async_copy(
            table_hbm.at[idx_v.at[pl.ds(c * _CHUNK, _CHUNK)]],
            rows_v.at[b], sem_g.at[b])

    def writeback(c, b):
        return pltpu.make_async_copy(
            rows_v.at[b], out_hbm.at[pl.ds(base + c * _CHUNK, _CHUNK)],
            sem_o.at[b])

    for b in range(_NBUF):
        gather(b, b).start()
    for b in range(_NBUF):
        gather(b, b).wait()
        writeback(b, b).start()

    def step(g, carry):
        for b in range(_NBUF):
            c = _NBUF * g + b
            writeback(c - _NBUF, b).wait()
            gather(c, b).start()
            gather(c, b).wait()
            writeback(c, b).start()
        return carry

    lax.fori_loop(1, nchunks // _NBUF, step, 0)

    for b in range(_NBUF):
        writeback(nchunks - _NBUF + b, b).wait()


def kernel(x_ids, table):
    b, h = x_ids.shape
    n = b * h
    idx = x_ids.reshape(n).astype(jnp.int32)
    table_wide = lax.optimization_barrier(
        table.reshape(table.shape[0] * _EMBED // 128, 128))
    table_lin = table_wide.reshape(table.shape[0], _EMBED)
    run = pl.kernel(
        _emb_body,
        mesh=plsc.VectorSubcoreMesh(core_axis_name="c", subcore_axis_name="s"),
        out_type=jax.ShapeDtypeStruct((n, _EMBED), jnp.float32),
        scratch_types=[
            pltpu.VMEM((n // _NW,), jnp.int32),
            pltpu.VMEM((_NBUF, _CHUNK, _EMBED), jnp.float32),
            pltpu.SemaphoreType.DMA((_NBUF,)),
            pltpu.SemaphoreType.DMA((_NBUF,)),
        ],
        compiler_params=pltpu.CompilerParams(use_tc_tiling_on_sc=False),
    )
    out = run(idx, table_lin)
    out_wide = lax.optimization_barrier(out.reshape(n * _EMBED // 128, 128))
    return out_wide.reshape(b, h, _EMBED)
